# baseline (speedup 1.0000x reference)
import functools

import jax
import jax.numpy as jnp
from jax import lax
from jax.experimental import pallas as pl
from jax.experimental.pallas import tpu as pltpu
from jax.experimental.pallas import tpu_sc as plsc

_RES = 0.8
_POINT_Z = 0.8
_NB = 20
_NSPLIT = 16
_K = 3
_NC = 2
_NS = 16


def _box_mask(bx, H, W, row0):
    c = jnp.cos(bx[:, 6])
    s = jnp.sin(bx[:, 6])
    k1 = c * bx[:, 0] + s * bx[:, 1]
    k2 = -s * bx[:, 0] + c * bx[:, 1]
    adx2 = jnp.abs(bx[:, 3]) * 0.5
    ady2 = jnp.abs(bx[:, 4]) * 0.5
    adz2 = jnp.abs(bx[:, 5]) * 0.5
    zok = jnp.abs(_POINT_Z - bx[:, 2]) <= adz2
    adx2 = jnp.where(zok, adx2, -1.0)

    xs_r = (jax.lax.broadcasted_iota(jnp.int32, (H, 1), 0).astype(jnp.float32)
            + (row0 - 128.0)) * _RES
    ys_c = (jax.lax.broadcasted_iota(jnp.int32, (1, W), 1).astype(jnp.float32)
            - W / 2.0) * _RES

    mask = None
    for nb in range(_NB):
        ax = c[nb] * xs_r - k1[nb]
        bxv = s[nb] * ys_c
        ay = -s[nb] * xs_r - k2[nb]
        byv = c[nb] * ys_c
        ins = (jnp.abs(ax + bxv) <= adx2[nb]) \
            & (jnp.abs(ay + byv) <= ady2[nb])
        mask = ins if mask is None else (mask | ins)
    return mask.astype(jnp.float32)


def _iou_from_grids(pred_g, orig_g, maskf):
    inter = jnp.sum(pred_g * orig_g * maskf, keepdims=True)
    union = jnp.sum(jnp.maximum(pred_g, orig_g) * maskf, keepdims=True)
    return inter / (union + 1e-6)


def _sc_sums(added_points, original_points):
    B, T, H, W = added_points.shape
    rows = H // (_NC * _NS)
    mesh = plsc.VectorSubcoreMesh(core_axis_name="c", subcore_axis_name="s")

    @functools.partial(
        pl.kernel, mesh=mesh,
        out_type=jax.ShapeDtypeStruct((_K, 2, H, W), jnp.float32),
        scratch_types=[
            pltpu.VMEM((T, rows, W), jnp.float32),
            pltpu.VMEM((rows, W), jnp.float32),
        ],
    )
    def _sum_kernel(a_hbm, o_hbm, out_hbm, fr_v, acc_v):
        wid = lax.axis_index("s") * _NC + lax.axis_index("c")
        r0 = wid * rows
        for b in range(_K):
            for which in range(2):
                src = a_hbm if which == 0 else o_hbm
                t0 = 0 if which == 0 else 1
                pltpu.sync_copy(
                    src.at[b, pl.ds(t0, T), pl.ds(r0, rows), :], fr_v)

                @pl.loop(0, rows * (W // 16))
                def _acc(i):
                    r = i // (W // 16)
                    o = (i % (W // 16)) * 16
                    v = fr_v[0, r, pl.ds(o, 16)]
                    for t in range(1, T):
                        v = v + fr_v[t, r, pl.ds(o, 16)]
                    acc_v[r, pl.ds(o, 16)] = v

                pltpu.sync_copy(
                    acc_v, out_hbm.at[b, which, pl.ds(r0, rows), :])

    return _sum_kernel(added_points, original_points)


def _loss_kernel(boxes_ref, *refs):
    a_refs = refs[:_NSPLIT]
    o_refs = refs[_NSPLIT:2 * _NSPLIT]
    out_ref = refs[2 * _NSPLIT]
    Hs = a_refs[0].shape[2]
    W = a_refs[0].shape[3]

    bx = boxes_ref[0]
    inter = None
    union = None
    for i in range(_NSPLIT):
        pred = jnp.sum(a_refs[i][0], axis=0)
        orig = jnp.sum(o_refs[i][0, 1:], axis=0)
        pred_g = (pred > 0.0).astype(jnp.float32)
        orig_g = (orig > 0.0).astype(jnp.float32)
        maskf = _box_mask(bx, Hs, W, float(i * Hs))
        i_h = jnp.sum(pred_g * orig_g * maskf, keepdims=True)
        u_h = jnp.sum(jnp.maximum(pred_g, orig_g) * maskf, keepdims=True)
        inter = i_h if inter is None else inter + i_h
        union = u_h if union is None else union + u_h

    iou = inter / (union + 1e-6)
    out_ref[...] = iou[None]


def _finish_kernel(boxes_ref, grids_ref, out_ref):
    H, W = grids_ref.shape[2], grids_ref.shape[3]
    pred_g = (grids_ref[0, 0] > 0.0).astype(jnp.float32)
    orig_g = (grids_ref[0, 1] > 0.0).astype(jnp.float32)
    maskf = _box_mask(boxes_ref[0], H, W, 0.0)
    out_ref[...] = _iou_from_grids(pred_g, orig_g, maskf)[None]


def kernel(added_points, original_points, boxes, tf_ego):
    B, T, H, W = added_points.shape
    boxes_p = jnp.zeros((B, 32, 8), dtype=jnp.float32)
    boxes_p = boxes_p.at[:, : boxes.shape[1], :7].set(boxes)
    Hs = H // _NSPLIT

    sc_grids = _sc_sums(added_points, original_points)

    def _a_spec(i):
        return pl.BlockSpec((1, T, Hs, W), lambda b, i=i: (b + _K, 0, i, 0))

    def _o_spec(i):
        return pl.BlockSpec((1, T + 1, Hs, W),
                            lambda b, i=i: (b + _K, 0, i, 0))

    out = pl.pallas_call(
        _loss_kernel,
        grid=(B - _K,),
        in_specs=[pl.BlockSpec((1, 32, 8), lambda b: (b + _K, 0, 0))]
        + [_a_spec(i) for i in range(_NSPLIT)]
        + [_o_spec(i) for i in range(_NSPLIT)],
        out_specs=pl.BlockSpec((1, 1, 1), lambda b: (b, 0, 0)),
        out_shape=jax.ShapeDtypeStruct((B - _K, 1, 1), jnp.float32),
        compiler_params=pltpu.CompilerParams(
            dimension_semantics=("arbitrary",),
            vmem_limit_bytes=110 * 1024 * 1024,
        ),
    )(boxes_p, *([added_points] * _NSPLIT), *([original_points] * _NSPLIT))

    fin = pl.pallas_call(
        _finish_kernel,
        grid=(_K,),
        in_specs=[
            pl.BlockSpec((1, 32, 8), lambda b: (b, 0, 0)),
            pl.BlockSpec((1, 2, H, W), lambda b: (b, 0, 0, 0)),
        ],
        out_specs=pl.BlockSpec((1, 1, 1), lambda b: (b, 0, 0)),
        out_shape=jax.ShapeDtypeStruct((_K, 1, 1), jnp.float32),
    )(boxes_p, sc_grids)

    return (jnp.sum(out) + jnp.sum(fin)) / B
